# Initial kernel scaffold; baseline (speedup 1.0000x reference)
import jax
import jax.numpy as jnp
from jax.experimental import pallas as pl


def kernel(masks, labels, scores):
    raise NotImplementedError("write your pallas kernel here")


# speedup vs baseline: 1.0960x; 1.0960x over previous
import functools

import jax
import jax.numpy as jnp
from jax import lax
from jax.experimental import pallas as pl
from jax.experimental.pallas import tpu as pltpu
from jax.experimental.pallas import tpu_sc as plsc

_NUM_CLASSES = 80
_N = 128
_H = 512
_W = 512
_NC = 2
_NS = 16
_NW = _NC * _NS
_S = 512
_ROWS_PER_W = _H // _NW
_L = 16


def _sc_body(masks_hbm, labels_hbm, scores_hbm, out_hbm,
             lab_v, sco_v, in_v, acc_v, sem):
    wid = lax.axis_index("s") * _NC + lax.axis_index("c")

    pltpu.sync_copy(labels_hbm, lab_v.at[pl.ds(0, _N)])
    pltpu.sync_copy(scores_hbm, sco_v.at[pl.ds(0, _N)])

    def block_body(b, carry):
        cidx = wid * _ROWS_PER_W + b

        pltpu.async_copy(masks_hbm.at[:, cidx, :], in_v, sem).wait()

        def zero_c(c, carry2):
            for j in range(_S // _L):
                acc_v[c, pl.ds(j * _L, _L)] = jnp.zeros((_L,), jnp.float32)
            return carry2
        lax.fori_loop(0, _NUM_CLASSES, zero_c, 0)

        def inst(i, carry2):
            gate = sco_v[pl.ds(i, _L)][0] >= 0.5

            @pl.when(gate)
            def _():
                lab = lab_v[pl.ds(i, _L)][0]
                for j in range(_S // _L):
                    m = in_v[i, pl.ds(j * _L, _L)]
                    v = jnp.where(m > 0.5,
                                  jnp.float32(1.0), jnp.float32(0.0))
                    plsc.addupdate(acc_v.at[lab, pl.ds(j * _L, _L)], v)
            return carry2
        lax.fori_loop(0, _N, inst, 0)

        def binarize_c(c, carry2):
            for j in range(_S // _L):
                v = acc_v[c, pl.ds(j * _L, _L)]
                acc_v[c, pl.ds(j * _L, _L)] = jnp.where(
                    v > 0.0, jnp.float32(1.0), jnp.float32(0.0))
            return carry2
        lax.fori_loop(0, _NUM_CLASSES, binarize_c, 0)

        pltpu.sync_copy(acc_v, out_hbm.at[:, cidx, :])
        return carry

    lax.fori_loop(0, _ROWS_PER_W, block_body, 0)


def _seg_any_sc(masks3, labels, scores):
    mesh = plsc.VectorSubcoreMesh(core_axis_name="c", subcore_axis_name="s",
                                  num_cores=_NC, num_subcores=_NS)
    return pl.kernel(
        _sc_body,
        out_type=jax.ShapeDtypeStruct((_NUM_CLASSES, _H, _W), jnp.float32),
        mesh=mesh,
        scratch_types=[
            pltpu.VMEM((_N + _L,), jnp.int32),
            pltpu.VMEM((_N + _L,), jnp.float32),
            pltpu.VMEM((_N, _S), jnp.float32),
            pltpu.VMEM((_NUM_CLASSES, _S), jnp.float32),
            pltpu.SemaphoreType.DMA,
        ],
    )(masks3, labels, scores)


def _labels_body(lab_row_ref, lab_col_ref, out_ref):
    C = _NUM_CLASSES
    lab_row = lab_row_ref[...]
    lab_col = lab_col_ref[...]

    cls_cn = lax.broadcasted_iota(jnp.int32, (C, _N), 0)
    presence_col = jnp.max(
        (cls_cn == jnp.broadcast_to(lab_row, (C, _N))).astype(jnp.int32),
        axis=1, keepdims=True)

    cls_nc = lax.broadcasted_iota(jnp.int32, (_N, C), 1)
    presence_row = jnp.max(
        (cls_nc == jnp.broadcast_to(lab_col, (_N, C))).astype(jnp.int32),
        axis=0, keepdims=True)

    r_iota = lax.broadcasted_iota(jnp.int32, (C, C), 0)
    k_iota = lax.broadcasted_iota(jnp.int32, (C, C), 1)
    tri = (k_iota <= r_iota).astype(jnp.int32)
    count_col = jnp.sum(tri * jnp.broadcast_to(presence_row, (C, C)),
                        axis=1, keepdims=True)

    total = jnp.sum(presence_col)

    match = jnp.logical_and(
        jnp.broadcast_to(count_col, (C, C)) - 1 == k_iota,
        jnp.broadcast_to(presence_col, (C, C)) == 1)
    out_row = jnp.sum(jnp.where(match, r_iota, 0),
                      axis=0, keepdims=True)
    j_row = lax.broadcasted_iota(jnp.int32, (1, C), 1)
    out_ref[...] = jnp.where(j_row >= total, jnp.int32(C), out_row)


def _new_labels_tc(labels):
    lab_row = labels.reshape(1, _N)
    lab_col = labels.reshape(_N, 1)
    out = pl.pallas_call(
        _labels_body,
        out_shape=jax.ShapeDtypeStruct((1, _NUM_CLASSES), jnp.int32),
    )(lab_row, lab_col)
    return out.reshape(_NUM_CLASSES)


@jax.jit
def kernel(masks, labels, scores):
    masks3 = masks.reshape(_N, _H, _W)
    labels = labels.astype(jnp.int32)
    counts = _seg_any_sc(masks3, labels, scores)
    sem_masks = counts.astype(bool)
    new_labels = _new_labels_tc(labels)
    return sem_masks, new_labels


# speedup vs baseline: 1.4374x; 1.3115x over previous
import functools

import jax
import jax.numpy as jnp
from jax import lax
from jax.experimental import pallas as pl
from jax.experimental.pallas import tpu as pltpu
from jax.experimental.pallas import tpu_sc as plsc

_NUM_CLASSES = 80
_N = 128
_H = 512
_W = 512
_NC = 2
_NS = 16
_NW = _NC * _NS
_S = 512
_ROWS_PER_W = _H // _NW
_L = 16
_CH = 16
_NCHUNK = _N // _CH


def _sc_body(masks_hbm, labels_hbm, scores_hbm, out_hbm,
             lab_v, sco_v, gidx_v, glab_v, idx2_v,
             in_a, in_b, acc_a, acc_b,
             sem_a, sem_b, sem_oa, sem_ob):
    wid = lax.axis_index("s") * _NC + lax.axis_index("c")

    pltpu.sync_copy(labels_hbm, lab_v.at[pl.ds(0, _N)])
    pltpu.sync_copy(scores_hbm, sco_v.at[pl.ds(0, _N)])

    for k in range((_N + _L) // _L):
        gidx_v[pl.ds(k * _L, _L)] = jnp.zeros((_L,), jnp.int32)

    def compact(i, off):
        gate = sco_v[pl.ds(i, _L)][0] >= 0.5

        @pl.when(gate)
        def _():
            gidx_v[pl.ds(off, _L)] = jnp.full((_L,), i, jnp.int32)
            glab_v[pl.ds(off, _L)] = jnp.full(
                (_L,), lab_v[pl.ds(i, _L)][0], jnp.int32)
        return off + jnp.where(gate, jnp.int32(1), jnp.int32(0))
    ngated = lax.fori_loop(0, _N, compact, jnp.int32(0))
    nchunks = (ngated + (_CH - 1)) // _CH

    def compute_chunk(k, buf, acc):
        cnt = jnp.minimum(jnp.int32(_CH), ngated - k * _CH)

        def inst(r, carry):
            lab = glab_v[pl.ds(k * _CH + r, _L)][0]
            for j in range(_S // _L):
                m = buf[r, pl.ds(j * _L, _L)]
                v = jnp.where(m > 0.5, jnp.float32(1.0), jnp.float32(0.0))
                plsc.addupdate(acc.at[lab, pl.ds(j * _L, _L)], v)
            return carry
        lax.fori_loop(0, cnt, inst, 0)

    def do_block(b, acc, sem_o, sb):
        cidx = wid * _ROWS_PER_W + b

        for k in range(_NCHUNK):
            idx2_v[k, :] = gidx_v[pl.ds(k * _L, _L)] * jnp.int32(_H) + cidx

        @pl.when(jnp.int32(0) < nchunks)
        def _():
            pltpu.async_copy(masks_hbm.at[idx2_v.at[0]], in_a, sem_a)

        @pl.when(sb >= 1)
        def _():
            pltpu.make_async_copy(acc, out_hbm.at[:, 0, :], sem_o).wait()

        def zero_c(c, carry2):
            for j in range(_S // _L):
                acc[c, pl.ds(j * _L, _L)] = jnp.zeros((_L,), jnp.float32)
            return carry2
        lax.fori_loop(0, _NUM_CLASSES, zero_c, 0)

        for k in range(_NCHUNK):
            buf = in_a if k % 2 == 0 else in_b
            sem = sem_a if k % 2 == 0 else sem_b
            nbuf = in_b if k % 2 == 0 else in_a
            nsem = sem_b if k % 2 == 0 else sem_a

            @pl.when(jnp.int32(k) < nchunks)
            def _(k=k, buf=buf, sem=sem, nbuf=nbuf, nsem=nsem):
                @pl.when(jnp.int32(k + 1) < nchunks)
                def _():
                    pltpu.async_copy(masks_hbm.at[idx2_v.at[k + 1]],
                                     nbuf, nsem)
                pltpu.make_async_copy(masks_hbm.at[idx2_v.at[k]],
                                      buf, sem).wait()
                compute_chunk(jnp.int32(k), buf, acc)

        def binarize_c(c, carry2):
            for j in range(_S // _L):
                v = acc[c, pl.ds(j * _L, _L)]
                acc[c, pl.ds(j * _L, _L)] = jnp.where(
                    v > 0.0, jnp.float32(1.0), jnp.float32(0.0))
            return carry2
        lax.fori_loop(0, _NUM_CLASSES, binarize_c, 0)

        pltpu.async_copy(acc, out_hbm.at[:, cidx, :], sem_o)

    def superblock(sb, carry):
        do_block(sb * 2, acc_a, sem_oa, sb)
        do_block(sb * 2 + 1, acc_b, sem_ob, sb)
        return carry
    lax.fori_loop(0, _ROWS_PER_W // 2, superblock, 0)

    pltpu.make_async_copy(acc_a, out_hbm.at[:, 0, :], sem_oa).wait()
    pltpu.make_async_copy(acc_b, out_hbm.at[:, 0, :], sem_ob).wait()


def _seg_any_sc(masks2, labels, scores):
    mesh = plsc.VectorSubcoreMesh(core_axis_name="c", subcore_axis_name="s",
                                  num_cores=_NC, num_subcores=_NS)
    return pl.kernel(
        _sc_body,
        out_type=jax.ShapeDtypeStruct((_NUM_CLASSES, _H, _W), jnp.float32),
        mesh=mesh,
        scratch_types=[
            pltpu.VMEM((_N + _L,), jnp.int32),
            pltpu.VMEM((_N + _L,), jnp.float32),
            pltpu.VMEM((_N + _L,), jnp.int32),
            pltpu.VMEM((_N + _L,), jnp.int32),
            pltpu.VMEM((_NCHUNK, _CH), jnp.int32),
            pltpu.VMEM((_CH, _S), jnp.float32),
            pltpu.VMEM((_CH, _S), jnp.float32),
            pltpu.VMEM((_NUM_CLASSES, _S), jnp.float32),
            pltpu.VMEM((_NUM_CLASSES, _S), jnp.float32),
            pltpu.SemaphoreType.DMA,
            pltpu.SemaphoreType.DMA,
            pltpu.SemaphoreType.DMA,
            pltpu.SemaphoreType.DMA,
        ],
    )(masks2, labels, scores)


def _labels_body(lab_row_ref, lab_col_ref, out_ref):
    C = _NUM_CLASSES
    lab_row = lab_row_ref[...]
    lab_col = lab_col_ref[...]

    cls_cn = lax.broadcasted_iota(jnp.int32, (C, _N), 0)
    presence_col = jnp.max(
        (cls_cn == jnp.broadcast_to(lab_row, (C, _N))).astype(jnp.int32),
        axis=1, keepdims=True)

    cls_nc = lax.broadcasted_iota(jnp.int32, (_N, C), 1)
    presence_row = jnp.max(
        (cls_nc == jnp.broadcast_to(lab_col, (_N, C))).astype(jnp.int32),
        axis=0, keepdims=True)

    r_iota = lax.broadcasted_iota(jnp.int32, (C, C), 0)
    k_iota = lax.broadcasted_iota(jnp.int32, (C, C), 1)
    tri = (k_iota <= r_iota).astype(jnp.int32)
    count_col = jnp.sum(tri * jnp.broadcast_to(presence_row, (C, C)),
                        axis=1, keepdims=True)

    total = jnp.sum(presence_col)

    match = jnp.logical_and(
        jnp.broadcast_to(count_col, (C, C)) - 1 == k_iota,
        jnp.broadcast_to(presence_col, (C, C)) == 1)
    out_row = jnp.sum(jnp.where(match, r_iota, 0),
                      axis=0, keepdims=True)
    j_row = lax.broadcasted_iota(jnp.int32, (1, C), 1)
    out_ref[...] = jnp.where(j_row >= total, jnp.int32(C), out_row)


def _new_labels_tc(labels):
    lab_row = labels.reshape(1, _N)
    lab_col = labels.reshape(_N, 1)
    out = pl.pallas_call(
        _labels_body,
        out_shape=jax.ShapeDtypeStruct((1, _NUM_CLASSES), jnp.int32),
    )(lab_row, lab_col)
    return out.reshape(_NUM_CLASSES)


@jax.jit
def kernel(masks, labels, scores):
    masks2 = masks.reshape(_N * _H, _W)
    labels = labels.astype(jnp.int32)
    counts = _seg_any_sc(masks2, labels, scores)
    sem_masks = counts.astype(bool)
    new_labels = _new_labels_tc(labels)
    return sem_masks, new_labels


# speedup vs baseline: 1.5427x; 1.0733x over previous
import functools

import jax
import jax.numpy as jnp
from jax import lax
from jax.experimental import pallas as pl
from jax.experimental.pallas import tpu as pltpu
from jax.experimental.pallas import tpu_sc as plsc

_NUM_CLASSES = 80
_N = 128
_H = 512
_W = 512
_NC = 2
_NS = 16
_NW = _NC * _NS
_S = 512
_ROWS_PER_W = _H // _NW
_L = 16
_CH = 16
_NCHUNK = _N // _CH


def _sc_body(masks_hbm, labels_hbm, scores_hbm, out_hbm,
             lab_v, sco_v, gidx_v, glab_v, idx2_v,
             in_a, in_b, acc_a, acc_b,
             sem_a, sem_b, sem_oa, sem_ob):
    wid = lax.axis_index("s") * _NC + lax.axis_index("c")

    pltpu.sync_copy(labels_hbm, lab_v.at[pl.ds(0, _N)])
    pltpu.sync_copy(scores_hbm, sco_v.at[pl.ds(0, _N)])

    for k in range((_N + _L) // _L):
        gidx_v[pl.ds(k * _L, _L)] = jnp.zeros((_L,), jnp.int32)
    def compact(i, off):
        gate = sco_v[pl.ds(i, _L)][0] >= 0.5

        @pl.when(gate)
        def _():
            lab = lab_v[pl.ds(i, _L)][0]
            gidx_v[pl.ds(off, _L)] = jnp.full((_L,), i, jnp.int32)
            glab_v[pl.ds(off, _L)] = jnp.full((_L,), lab, jnp.int32)
        return off + jnp.where(gate, jnp.int32(1), jnp.int32(0))
    ngated = lax.fori_loop(0, _N, compact, jnp.int32(0))
    nchunks = (ngated + (_CH - 1)) // _CH

    def zero_all(c, carry):
        for j in range(_S // _L):
            acc_a[c, pl.ds(j * _L, _L)] = jnp.zeros((_L,), jnp.float32)
            acc_b[c, pl.ds(j * _L, _L)] = jnp.zeros((_L,), jnp.float32)
        return carry
    lax.fori_loop(0, _NUM_CLASSES, zero_all, 0)

    def compute_chunk(k, buf, acc):
        cnt = jnp.minimum(jnp.int32(_CH), ngated - k * _CH)

        def inst(r, carry):
            lab = glab_v[pl.ds(k * _CH + r, _L)][0]
            for j in range(_S // _L):
                m = buf[r, pl.ds(j * _L, _L)]
                v = jnp.where(m > 0.5, jnp.float32(1.0), jnp.float32(0.0))
                plsc.addupdate(acc.at[lab, pl.ds(j * _L, _L)], v)
            return carry
        lax.fori_loop(0, cnt, inst, 0)

    def do_block(b, acc, sem_o, sb):
        cidx = wid * _ROWS_PER_W + b

        for k in range(_NCHUNK):
            idx2_v[k, :] = gidx_v[pl.ds(k * _L, _L)] * jnp.int32(_H) + cidx

        @pl.when(jnp.int32(0) < nchunks)
        def _():
            pltpu.async_copy(masks_hbm.at[idx2_v.at[0]], in_a, sem_a)

        @pl.when(sb >= 1)
        def _():
            pltpu.make_async_copy(acc, out_hbm.at[:, 0, :], sem_o).wait()

        def zero_c(c, carry2):
            for j in range(_S // _L):
                acc[c, pl.ds(j * _L, _L)] = jnp.zeros((_L,), jnp.float32)
            return carry2
        lax.fori_loop(0, _NUM_CLASSES, zero_c, 0)

        for k in range(_NCHUNK):
            buf = in_a if k % 2 == 0 else in_b
            sem = sem_a if k % 2 == 0 else sem_b
            nbuf = in_b if k % 2 == 0 else in_a
            nsem = sem_b if k % 2 == 0 else sem_a

            @pl.when(jnp.int32(k) < nchunks)
            def _(k=k, buf=buf, sem=sem, nbuf=nbuf, nsem=nsem):
                @pl.when(jnp.int32(k + 1) < nchunks)
                def _():
                    pltpu.async_copy(masks_hbm.at[idx2_v.at[k + 1]],
                                     nbuf, nsem)
                pltpu.make_async_copy(masks_hbm.at[idx2_v.at[k]],
                                      buf, sem).wait()
                compute_chunk(jnp.int32(k), buf, acc)

        pltpu.async_copy(acc, out_hbm.at[:, cidx, :], sem_o)

    def superblock(sb, carry):
        do_block(sb * 2, acc_a, sem_oa, sb)
        do_block(sb * 2 + 1, acc_b, sem_ob, sb)
        return carry
    lax.fori_loop(0, _ROWS_PER_W // 2, superblock, 0)

    pltpu.make_async_copy(acc_a, out_hbm.at[:, 0, :], sem_oa).wait()
    pltpu.make_async_copy(acc_b, out_hbm.at[:, 0, :], sem_ob).wait()


def _seg_any_sc(masks2, labels, scores):
    mesh = plsc.VectorSubcoreMesh(core_axis_name="c", subcore_axis_name="s",
                                  num_cores=_NC, num_subcores=_NS)
    return pl.kernel(
        _sc_body,
        out_type=jax.ShapeDtypeStruct((_NUM_CLASSES, _H, _W), jnp.float32),
        mesh=mesh,
        scratch_types=[
            pltpu.VMEM((_N + _L,), jnp.int32),
            pltpu.VMEM((_N + _L,), jnp.float32),
            pltpu.VMEM((_N + _L,), jnp.int32),
            pltpu.VMEM((_N + _L,), jnp.int32),
            pltpu.VMEM((_NCHUNK, _CH), jnp.int32),
            pltpu.VMEM((_CH, _S), jnp.float32),
            pltpu.VMEM((_CH, _S), jnp.float32),
            pltpu.VMEM((_NUM_CLASSES, _S), jnp.float32),
            pltpu.VMEM((_NUM_CLASSES, _S), jnp.float32),
            pltpu.SemaphoreType.DMA,
            pltpu.SemaphoreType.DMA,
            pltpu.SemaphoreType.DMA,
            pltpu.SemaphoreType.DMA,
        ],
    )(masks2, labels, scores)


def _labels_body(lab_row_ref, lab_col_ref, out_ref):
    C = _NUM_CLASSES
    lab_row = lab_row_ref[...]
    lab_col = lab_col_ref[...]

    cls_cn = lax.broadcasted_iota(jnp.int32, (C, _N), 0)
    presence_col = jnp.max(
        (cls_cn == jnp.broadcast_to(lab_row, (C, _N))).astype(jnp.int32),
        axis=1, keepdims=True)

    cls_nc = lax.broadcasted_iota(jnp.int32, (_N, C), 1)
    presence_row = jnp.max(
        (cls_nc == jnp.broadcast_to(lab_col, (_N, C))).astype(jnp.int32),
        axis=0, keepdims=True)

    r_iota = lax.broadcasted_iota(jnp.int32, (C, C), 0)
    k_iota = lax.broadcasted_iota(jnp.int32, (C, C), 1)
    tri = (k_iota <= r_iota).astype(jnp.int32)
    count_col = jnp.sum(tri * jnp.broadcast_to(presence_row, (C, C)),
                        axis=1, keepdims=True)

    total = jnp.sum(presence_col)

    match = jnp.logical_and(
        jnp.broadcast_to(count_col, (C, C)) - 1 == k_iota,
        jnp.broadcast_to(presence_col, (C, C)) == 1)
    out_row = jnp.sum(jnp.where(match, r_iota, 0),
                      axis=0, keepdims=True)
    j_row = lax.broadcasted_iota(jnp.int32, (1, C), 1)
    out_ref[...] = jnp.where(j_row >= total, jnp.int32(C), out_row)


def _new_labels_tc(labels):
    lab_row = labels.reshape(1, _N)
    lab_col = labels.reshape(_N, 1)
    out = pl.pallas_call(
        _labels_body,
        out_shape=jax.ShapeDtypeStruct((1, _NUM_CLASSES), jnp.int32),
    )(lab_row, lab_col)
    return out.reshape(_NUM_CLASSES)


@jax.jit
def kernel(masks, labels, scores):
    masks2 = masks.reshape(_N * _H, _W)
    labels = labels.astype(jnp.int32)
    counts = _seg_any_sc(masks2, labels, scores)
    sem_masks = counts.astype(bool)
    new_labels = _new_labels_tc(labels)
    return sem_masks, new_labels
